# Initial kernel scaffold; baseline (speedup 1.0000x reference)
import jax
import jax.numpy as jnp
from jax.experimental import pallas as pl


def kernel(x, token_table, pos_table):
    raise NotImplementedError("write your pallas kernel here")


# speedup vs baseline: 1.2789x; 1.2789x over previous
import functools

import jax
import jax.numpy as jnp
from jax import lax
from jax.experimental import pallas as pl
from jax.experimental.pallas import tpu as pltpu
from jax.experimental.pallas import tpu_sc as plsc

_B = 4
_S = 2048
_D = 128
_BS = _B * _S

_info = plsc.get_sparse_core_info()
_NC = _info.num_cores
_NS = _info.num_subcores
_NW = _NC * _NS
_BPW = _BS // _NW
_IDX_ROWS = _BPW // 128
_LANES = 16
_CHUNKS = _D // _LANES


def _body(x_hbm, tok_hbm, pos_hbm, out_hbm, idx_v, rows_v, pos_v, sem):
    wid = lax.axis_index("s") * _NC + lax.axis_index("c")
    base = wid * _BPW

    pltpu.sync_copy(x_hbm.at[pl.ds(wid * _IDX_ROWS, _IDX_ROWS)], idx_v)

    handles = []
    for j in range(_IDX_ROWS):
        handles.append(
            pltpu.async_copy(
                tok_hbm.at[idx_v.at[j]],
                rows_v.at[pl.ds(j * 128, 128)],
                sem,
            )
        )
    pos_base = (wid % (_S // _BPW)) * _BPW
    pltpu.sync_copy(pos_hbm.at[pl.ds(pos_base, _BPW)], pos_v)
    for h in handles:
        h.wait()

    def add_row(r, carry):
        for c in range(_CHUNKS):
            sl = pl.ds(c * _LANES, _LANES)
            rows_v[r, sl] = rows_v[r, sl] + pos_v[r, sl]
        return carry

    lax.fori_loop(0, _BPW, add_row, 0)

    pltpu.sync_copy(rows_v, out_hbm.at[pl.ds(base, _BPW)])


@jax.jit
def _embed(x_flat, token_table, pos_table):
    mesh = plsc.VectorSubcoreMesh(core_axis_name="c", subcore_axis_name="s")
    k = functools.partial(
        pl.kernel,
        mesh=mesh,
        out_type=jax.ShapeDtypeStruct((_BS, _D), jnp.float32),
        scratch_types=[
            pltpu.VMEM((_IDX_ROWS, 128), jnp.int32),
            pltpu.VMEM((_BPW, _D), jnp.float32),
            pltpu.VMEM((_BPW, _D), jnp.float32),
            pltpu.SemaphoreType.DMA,
        ],
    )(_body)
    return k(x_flat, token_table, pos_table)


def kernel(x, token_table, pos_table):
    x_flat = x.reshape(_BS // 128, 128).astype(jnp.int32)
    out = _embed(x_flat, token_table, pos_table)
    return out.reshape(_B, _S, _D)


# speedup vs baseline: 1.3006x; 1.0170x over previous
import functools

import jax
import jax.numpy as jnp
from jax import lax
from jax.experimental import pallas as pl
from jax.experimental.pallas import tpu as pltpu
from jax.experimental.pallas import tpu_sc as plsc

_B = 4
_S = 2048
_D = 128
_BS = _B * _S

_info = plsc.get_sparse_core_info()
_NC = _info.num_cores
_NS = _info.num_subcores
_NW = _NC * _NS
_BPW = _BS // _NW
_IDX_ROWS = _BPW // 128
_LANES = 16
_CHUNKS = _D // _LANES


_NCHUNK = 4
_CH = _BPW // _NCHUNK


def _body(x_hbm, tok_hbm, pos_hbm, out_hbm, idx_v, rows_v, pos_v, gsem, wsem):
    wid = lax.axis_index("s") * _NC + lax.axis_index("c")
    base = wid * _BPW

    pltpu.sync_copy(x_hbm.at[pl.ds(wid * _IDX_ROWS, _IDX_ROWS)], idx_v)

    ghandles = []
    for k in range(_NCHUNK):
        j, off = divmod(k * _CH, 128)
        ghandles.append(
            pltpu.async_copy(
                tok_hbm.at[idx_v.at[j, pl.ds(off, _CH)]],
                rows_v.at[pl.ds(k * _CH, _CH)],
                gsem,
            )
        )
    pos_base = (wid % (_S // _BPW)) * _BPW
    pltpu.sync_copy(pos_hbm.at[pl.ds(pos_base, _BPW)], pos_v)

    def add_row(r, carry):
        for c in range(_CHUNKS):
            sl = pl.ds(c * _LANES, _LANES)
            rows_v[r, sl] = rows_v[r, sl] + pos_v[r, sl]
        return carry

    whandles = []
    for k in range(_NCHUNK):
        ghandles[k].wait()
        lax.fori_loop(k * _CH, (k + 1) * _CH, add_row, 0)
        whandles.append(
            pltpu.async_copy(
                rows_v.at[pl.ds(k * _CH, _CH)],
                out_hbm.at[pl.ds(base + k * _CH, _CH)],
                wsem,
            )
        )
    for h in whandles:
        h.wait()


@jax.jit
def _embed(x_flat, token_table, pos_table):
    mesh = plsc.VectorSubcoreMesh(core_axis_name="c", subcore_axis_name="s")
    k = functools.partial(
        pl.kernel,
        mesh=mesh,
        out_type=jax.ShapeDtypeStruct((_BS, _D), jnp.float32),
        scratch_types=[
            pltpu.VMEM((_IDX_ROWS, 128), jnp.int32),
            pltpu.VMEM((_BPW, _D), jnp.float32),
            pltpu.VMEM((_BPW, _D), jnp.float32),
            pltpu.SemaphoreType.DMA,
            pltpu.SemaphoreType.DMA,
        ],
    )(_body)
    return k(x_flat, token_table, pos_table)


def kernel(x, token_table, pos_table):
    x_flat = x.reshape(_BS // 128, 128).astype(jnp.int32)
    out = _embed(x_flat, token_table, pos_table)
    return out.reshape(_B, _S, _D)
